# Initial kernel scaffold; baseline (speedup 1.0000x reference)
import jax
import jax.numpy as jnp
from jax.experimental import pallas as pl


def kernel(node_ids, edge_index, ptr, emb, gin_w1, gin_b1, gin_lng, gin_lnb, gin_w2, gin_b2, seq_w1, seq_b1, seq_w2, seq_b2, norm_g, norm_b, ex_w1, ex_b1, ex_lng, ex_lnb, ex_w2, ex_b2):
    raise NotImplementedError("write your pallas kernel here")


# speedup vs baseline: 12.7793x; 12.7793x over previous
import functools
import math

import jax
import jax.numpy as jnp
from jax import lax
from jax.experimental import pallas as pl
from jax.experimental.pallas import tpu as pltpu
from jax.experimental.pallas import tpu_sc as plsc

N_NODES = 128
B = 256
N = B * N_NODES
E = 524288
D = 128

NC, NS = 2, 16
ITEMS = E + N
PER_TILE = ITEMS // NS
SUB = 2048
NSUB = PER_TILE // SUB
CHUNK_ROWS = N // (2 * NC)
CHUNK = CHUNK_ROWS * D
ROWS_PER_TILE = CHUNK_ROWS // NS
ZB = 16384


def _hist_body(src_hbm, dst_hbm, nid_hbm, c_hbm,
               nid_v, src_v, dst_v, idx_v, val_v, zero_v, shared):
    c = lax.axis_index("c")
    s = lax.axis_index("s")

    pltpu.sync_copy(nid_hbm, nid_v)

    z16 = jnp.zeros((16,), jnp.float32)

    def zb_body(i, _):
        zero_v[pl.ds(i * 16, 16)] = z16
        return 0

    lax.fori_loop(0, ZB // 16, zb_body, 0)

    tile_base = s * PER_TILE

    for chunk in range(2):
        base_bin = (c * 2 + chunk) * CHUNK

        for r in range(CHUNK // NS // ZB):
            pltpu.sync_copy(
                zero_v, shared.at[pl.ds(s * (CHUNK // NS) + r * ZB, ZB)])
        plsc.subcore_barrier()

        def sub_body(sb, _):
            ib = tile_base + sb * SUB
            pltpu.sync_copy(src_hbm.at[pl.ds(ib, SUB)], src_v)
            pltpu.sync_copy(dst_hbm.at[pl.ds(ib, SUB)], dst_v)

            def grp_body(g, _):
                row = g // 8
                col = (g % 8) * 16
                s16 = src_v[pl.ds(g * 16, 16)]
                d16 = dst_v[pl.ds(g * 16, 16)]
                n16 = plsc.load_gather(nid_v, [s16])
                bin_ = d16 * D + n16 - base_bin
                inr = (bin_ >= 0) & (bin_ < CHUNK)
                val_v[row, pl.ds(col, 16)] = jnp.where(inr, 1.0, 0.0)
                idx_v[row, pl.ds(col, 16)] = lax.bitwise_and(bin_, CHUNK - 1)
                return 0

            lax.fori_loop(0, SUB // 16, grp_body, 0)

            for r in range(SUB // 128):
                pltpu.sync_copy(val_v.at[r], shared.at[idx_v.at[r]], add=True)
            return 0

        lax.fori_loop(0, NSUB, sub_body, 0)
        plsc.subcore_barrier()

        out0 = base_bin + s * (CHUNK // NS)
        pltpu.sync_copy(shared.at[pl.ds(s * (CHUNK // NS), CHUNK // NS)],
                        c_hbm.at[pl.ds(out0, CHUNK // NS)])


def _build_counts(src2, dst2, node_ids):
    mesh = plsc.VectorSubcoreMesh(core_axis_name="c", subcore_axis_name="s")
    return pl.kernel(
        _hist_body,
        out_type=jax.ShapeDtypeStruct((N * D,), jnp.float32),
        mesh=mesh,
        compiler_params=pltpu.CompilerParams(needs_layout_passes=False),
        scratch_types=[
            pltpu.VMEM((N,), jnp.int32),
            pltpu.VMEM((SUB,), jnp.int32),
            pltpu.VMEM((SUB,), jnp.int32),
            pltpu.VMEM((SUB // 128, 128), jnp.int32),
            pltpu.VMEM((SUB // 128, 128), jnp.float32),
            pltpu.VMEM((ZB,), jnp.float32),
            pltpu.VMEM_SHARED((CHUNK,), jnp.float32),
        ],
    )(src2, dst2, node_ids)


BLK = 512
G_PER_BLK = BLK // N_NODES
GRID = N // BLK


def _ln(h, g, b):
    m = jnp.mean(h, axis=-1, keepdims=True)
    v = jnp.mean((h - m) ** 2, axis=-1, keepdims=True)
    return (h - m) * lax.rsqrt(v + 1e-5) * g + b


def _dense_body(c_ref, emb_ref, w1_ref, b1_ref, lng_ref, lnb_ref,
                w2_ref, b2_ref, sw1_ref, sb1_ref, sw2_ref, sb2_ref,
                ng_ref, nb_ref, sums_ref, dp_ref):
    cnt = c_ref[:]
    m = jnp.dot(emb_ref[:], w1_ref[:], preferred_element_type=jnp.float32)
    h = jnp.dot(cnt, m, preferred_element_type=jnp.float32) + b1_ref[:]
    h = _ln(h, lng_ref[:], lnb_ref[:])
    h = jnp.maximum(h, 0.0)
    h = jnp.dot(h, w2_ref[:], preferred_element_type=jnp.float32) + b2_ref[:]
    h = jnp.dot(h, sw1_ref[:], preferred_element_type=jnp.float32) + sb1_ref[:]
    h = jnp.maximum(h, 0.0)
    h = jnp.dot(h, sw2_ref[:], preferred_element_type=jnp.float32) + sb2_ref[:]
    x = _ln(h, ng_ref[:], nb_ref[:])

    scale = 1.0 / math.sqrt(D)
    for g in range(G_PER_BLK):
        xg = x[g * N_NODES:(g + 1) * N_NODES, :]
        sums_ref[0, g, :] = jnp.sum(xg, axis=0)
        dp_ref[g, :, :] = lax.dot_general(
            xg, xg, (((1,), (1,)), ((), ())),
            preferred_element_type=jnp.float32) * scale


def _dense_stage(counts, emb, w1, b1, lng, lnb, w2, b2,
                 sw1, sb1, sw2, sb2, ng, nb):
    wspec = pl.BlockSpec((D, D), lambda i: (0, 0))
    bspec = pl.BlockSpec((1, D), lambda i: (0, 0))
    return pl.pallas_call(
        _dense_body,
        grid=(GRID,),
        in_specs=[
            pl.BlockSpec((BLK, D), lambda i: (i, 0)),
            wspec, wspec, bspec, bspec, bspec,
            wspec, bspec, wspec, bspec, wspec, bspec,
            bspec, bspec,
        ],
        out_specs=[
            pl.BlockSpec((1, G_PER_BLK, D), lambda i: (i, 0, 0)),
            pl.BlockSpec((G_PER_BLK, N_NODES, N_NODES), lambda i: (i, 0, 0)),
        ],
        out_shape=[
            jax.ShapeDtypeStruct((GRID, G_PER_BLK, D), jnp.float32),
            jax.ShapeDtypeStruct((B, N_NODES, N_NODES), jnp.float32),
        ],
    )(counts, emb, w1, b1, lng, lnb, w2, b2, sw1, sb1, sw2, sb2, ng, nb)


def _exit_body(sums_ref, w1_ref, b1_ref, lng_ref, lnb_ref, w2_ref, b2_ref,
               out_ref):
    means = sums_ref[:] * (1.0 / N_NODES)
    e = jnp.dot(means, w1_ref[:], preferred_element_type=jnp.float32) + b1_ref[:]
    e = _ln(e, lng_ref[:], lnb_ref[:])
    e = jnp.maximum(e, 0.0)
    out_ref[:] = jnp.dot(e, w2_ref[:],
                         preferred_element_type=jnp.float32) + b2_ref[:]


def _exit_stage(sums, w1, b1, lng, lnb, w2, b2):
    return pl.pallas_call(
        _exit_body,
        out_shape=jax.ShapeDtypeStruct((B, 1), jnp.float32),
    )(sums, w1, b1, lng, lnb, w2, b2)


def kernel(node_ids, edge_index, ptr, emb, gin_w1, gin_b1, gin_lng, gin_lnb,
           gin_w2, gin_b2, seq_w1, seq_b1, seq_w2, seq_b2, norm_g, norm_b,
           ex_w1, ex_b1, ex_lng, ex_lnb, ex_w2, ex_b2):
    del ptr
    node_ids = node_ids.astype(jnp.int32)
    self_idx = jnp.arange(N, dtype=jnp.int32)
    src2 = jnp.concatenate([edge_index[0].astype(jnp.int32), self_idx])
    dst2 = jnp.concatenate([edge_index[1].astype(jnp.int32), self_idx])

    counts = _build_counts(src2, dst2, node_ids).reshape(N, D)

    r2 = lambda v: v.reshape(1, D)
    sums, dp = _dense_stage(
        counts, emb, gin_w1, r2(gin_b1), r2(gin_lng), r2(gin_lnb),
        gin_w2, r2(gin_b2), seq_w1, r2(seq_b1), seq_w2, r2(seq_b2),
        r2(norm_g), r2(norm_b))

    exit_action = _exit_stage(
        sums.reshape(B, D), ex_w1, r2(ex_b1), r2(ex_lng), r2(ex_lnb), ex_w2,
        ex_b2.reshape(1, 1))

    i0, i1 = jnp.triu_indices(N_NODES, k=1)
    flat_idx = i0 * N_NODES + i1
    edge_actions = dp.reshape(B, N_NODES * N_NODES)[:, flat_idx]
    return jnp.concatenate([edge_actions, exit_action], axis=-1)


# speedup vs baseline: 16.2639x; 1.2727x over previous
import functools
import math

import jax
import jax.numpy as jnp
from jax import lax
from jax.experimental import pallas as pl
from jax.experimental.pallas import tpu as pltpu
from jax.experimental.pallas import tpu_sc as plsc

N_NODES = 128
B = 256
N = B * N_NODES
E = 524288
D = 128

NC, NS = 2, 16
ITEMS = E + N
PER_TILE = ITEMS // NS
SUB = 2048
NSUB = PER_TILE // SUB
CHUNK_ROWS = N // (2 * NC)
CHUNK = CHUNK_ROWS * D
ROWS_PER_TILE = CHUNK_ROWS // NS
ZB = 8192


def _hist_body(src_hbm, dst_hbm, nid_hbm, c_hbm,
               nid_v, src_v, dst_v, bin_v, idx_v, ones_v, zero_v, shared,
               esem, ssem):
    c = lax.axis_index("c")
    s = lax.axis_index("s")
    tile_base = s * PER_TILE
    slice0 = s * (CHUNK // NS)

    def edge_fetch(b):
        ib = tile_base + b * SUB
        return (pltpu.async_copy(src_hbm.at[pl.ds(ib, SUB)],
                                 src_v.at[b % 2], esem),
                pltpu.async_copy(dst_hbm.at[pl.ds(ib, SUB)],
                                 dst_v.at[b % 2], esem))

    edesc = edge_fetch(0)
    pltpu.sync_copy(nid_hbm, nid_v)

    z16 = jnp.zeros((16,), jnp.float32)
    o16 = jnp.ones((16,), jnp.float32)

    def zb_body(i, _):
        zero_v[pl.ds(i * 16, 16)] = z16
        return 0

    lax.fori_loop(0, ZB // 16, zb_body, 0)
    for i in range(8):
        ones_v[pl.ds(i * 16, 16)] = o16

    def zero_slice():
        descs = [pltpu.async_copy(
            zero_v, shared.at[pl.ds(slice0 + r * ZB, ZB)], esem)
            for r in range(CHUNK // NS // ZB)]
        for d in descs:
            d.wait()

    zero_slice()
    plsc.subcore_barrier()

    sdesc = [[], []]

    def fire_scatters(p):
        for r in range(SUB // 128):
            sdesc[p].append(pltpu.async_copy(
                ones_v,
                shared.at[plsc.Indices(idx_v.at[p, r], ignored_value=-1)],
                ssem, add=True))

    def drain_scatters(p):
        for d in sdesc[p]:
            d.wait()
        sdesc[p] = []

    base0 = c * 2 * CHUNK

    for b in range(NSUB):
        for d in edesc:
            d.wait()
        if b + 1 < NSUB:
            edesc = edge_fetch(b + 1)
        p = b % 2
        drain_scatters(p)
        bb = b * SUB

        def grp_body(g, _):
            row = g // 8
            col = (g % 8) * 16
            s16 = src_v[p, pl.ds(g * 16, 16)]
            d16 = dst_v[p, pl.ds(g * 16, 16)]
            w16 = plsc.load_gather(nid_v, [lax.shift_right_logical(s16, 2)])
            sh = lax.shift_left(lax.bitwise_and(s16, 3), 3)
            n16 = lax.bitwise_and(lax.shift_right_logical(w16, sh), 127)
            bin_ = lax.bitwise_or(lax.shift_left(d16, 7), n16)
            bin_v[pl.ds(bb + g * 16, 16)] = bin_
            rel = bin_ - base0
            inr = rel.astype(jnp.uint32) < CHUNK
            idx_v[p, row, pl.ds(col, 16)] = jnp.where(inr, rel, -1)
            return 0

        lax.fori_loop(0, SUB // 16, grp_body, 0)
        fire_scatters(p)

    drain_scatters(0)
    drain_scatters(1)
    plsc.subcore_barrier()
    pltpu.sync_copy(shared.at[pl.ds(slice0, CHUNK // NS)],
                    c_hbm.at[pl.ds(base0 + slice0, CHUNK // NS)])
    zero_slice()
    plsc.subcore_barrier()

    base1 = (c * 2 + 1) * CHUNK

    for b in range(NSUB):
        p = b % 2
        drain_scatters(p)
        bb = b * SUB

        def grp_body1(g, _):
            row = g // 8
            col = (g % 8) * 16
            bin_ = bin_v[pl.ds(bb + g * 16, 16)]
            rel = bin_ - base1
            inr = rel.astype(jnp.uint32) < CHUNK
            idx_v[p, row, pl.ds(col, 16)] = jnp.where(inr, rel, -1)
            return 0

        lax.fori_loop(0, SUB // 16, grp_body1, 0)
        fire_scatters(p)

    drain_scatters(0)
    drain_scatters(1)
    plsc.subcore_barrier()
    pltpu.sync_copy(shared.at[pl.ds(slice0, CHUNK // NS)],
                    c_hbm.at[pl.ds(base1 + slice0, CHUNK // NS)])


def _build_counts(src2, dst2, node_ids_packed):
    mesh = plsc.VectorSubcoreMesh(core_axis_name="c", subcore_axis_name="s")
    return pl.kernel(
        _hist_body,
        out_type=jax.ShapeDtypeStruct((N * D,), jnp.float32),
        mesh=mesh,
        compiler_params=pltpu.CompilerParams(needs_layout_passes=False),
        scratch_types=[
            pltpu.VMEM((N // 4,), jnp.int32),
            pltpu.VMEM((2, SUB), jnp.int32),
            pltpu.VMEM((2, SUB), jnp.int32),
            pltpu.VMEM((PER_TILE,), jnp.int32),
            pltpu.VMEM((2, SUB // 128, 128), jnp.int32),
            pltpu.VMEM((128,), jnp.float32),
            pltpu.VMEM((ZB,), jnp.float32),
            pltpu.VMEM_SHARED((CHUNK,), jnp.float32),
            pltpu.SemaphoreType.DMA,
            pltpu.SemaphoreType.DMA,
        ],
    )(src2, dst2, node_ids_packed)


BLK = 512
G_PER_BLK = BLK // N_NODES
GRID = N // BLK


def _ln(h, g, b):
    m = jnp.mean(h, axis=-1, keepdims=True)
    v = jnp.mean((h - m) ** 2, axis=-1, keepdims=True)
    return (h - m) * lax.rsqrt(v + 1e-5) * g + b


def _dense_body(c_ref, emb_ref, w1_ref, b1_ref, lng_ref, lnb_ref,
                w2_ref, b2_ref, sw1_ref, sb1_ref, sw2_ref, sb2_ref,
                ng_ref, nb_ref, sums_ref, dp_ref):
    cnt = c_ref[:]
    m = jnp.dot(emb_ref[:], w1_ref[:], preferred_element_type=jnp.float32)
    h = jnp.dot(cnt, m, preferred_element_type=jnp.float32) + b1_ref[:]
    h = _ln(h, lng_ref[:], lnb_ref[:])
    h = jnp.maximum(h, 0.0)
    h = jnp.dot(h, w2_ref[:], preferred_element_type=jnp.float32) + b2_ref[:]
    h = jnp.dot(h, sw1_ref[:], preferred_element_type=jnp.float32) + sb1_ref[:]
    h = jnp.maximum(h, 0.0)
    h = jnp.dot(h, sw2_ref[:], preferred_element_type=jnp.float32) + sb2_ref[:]
    x = _ln(h, ng_ref[:], nb_ref[:])

    scale = 1.0 / math.sqrt(D)
    for g in range(G_PER_BLK):
        xg = x[g * N_NODES:(g + 1) * N_NODES, :]
        sums_ref[0, g, :] = jnp.sum(xg, axis=0)
        dp_ref[g, :, :] = lax.dot_general(
            xg, xg, (((1,), (1,)), ((), ())),
            preferred_element_type=jnp.float32) * scale


def _dense_stage(counts, emb, w1, b1, lng, lnb, w2, b2,
                 sw1, sb1, sw2, sb2, ng, nb):
    wspec = pl.BlockSpec((D, D), lambda i: (0, 0))
    bspec = pl.BlockSpec((1, D), lambda i: (0, 0))
    return pl.pallas_call(
        _dense_body,
        grid=(GRID,),
        in_specs=[
            pl.BlockSpec((BLK, D), lambda i: (i, 0)),
            wspec, wspec, bspec, bspec, bspec,
            wspec, bspec, wspec, bspec, wspec, bspec,
            bspec, bspec,
        ],
        out_specs=[
            pl.BlockSpec((1, G_PER_BLK, D), lambda i: (i, 0, 0)),
            pl.BlockSpec((G_PER_BLK, N_NODES, N_NODES), lambda i: (i, 0, 0)),
        ],
        out_shape=[
            jax.ShapeDtypeStruct((GRID, G_PER_BLK, D), jnp.float32),
            jax.ShapeDtypeStruct((B, N_NODES, N_NODES), jnp.float32),
        ],
    )(counts, emb, w1, b1, lng, lnb, w2, b2, sw1, sb1, sw2, sb2, ng, nb)


def _exit_body(sums_ref, w1_ref, b1_ref, lng_ref, lnb_ref, w2_ref, b2_ref,
               out_ref):
    means = sums_ref[:] * (1.0 / N_NODES)
    e = jnp.dot(means, w1_ref[:], preferred_element_type=jnp.float32) + b1_ref[:]
    e = _ln(e, lng_ref[:], lnb_ref[:])
    e = jnp.maximum(e, 0.0)
    out_ref[:] = jnp.dot(e, w2_ref[:],
                         preferred_element_type=jnp.float32) + b2_ref[:]


def _exit_stage(sums, w1, b1, lng, lnb, w2, b2):
    return pl.pallas_call(
        _exit_body,
        out_shape=jax.ShapeDtypeStruct((B, 1), jnp.float32),
    )(sums, w1, b1, lng, lnb, w2, b2)


def kernel(node_ids, edge_index, ptr, emb, gin_w1, gin_b1, gin_lng, gin_lnb,
           gin_w2, gin_b2, seq_w1, seq_b1, seq_w2, seq_b2, norm_g, norm_b,
           ex_w1, ex_b1, ex_lng, ex_lnb, ex_w2, ex_b2):
    del ptr
    node_ids = node_ids.astype(jnp.int32)
    nid4 = node_ids.reshape(N // 4, 4)
    nid_packed = (nid4[:, 0] | (nid4[:, 1] << 8) | (nid4[:, 2] << 16)
                  | (nid4[:, 3] << 24))
    self_idx = jnp.arange(N, dtype=jnp.int32)
    src2 = jnp.concatenate([edge_index[0].astype(jnp.int32), self_idx])
    dst2 = jnp.concatenate([edge_index[1].astype(jnp.int32), self_idx])

    counts = _build_counts(src2, dst2, nid_packed).reshape(N, D)

    r2 = lambda v: v.reshape(1, D)
    sums, dp = _dense_stage(
        counts, emb, gin_w1, r2(gin_b1), r2(gin_lng), r2(gin_lnb),
        gin_w2, r2(gin_b2), seq_w1, r2(seq_b1), seq_w2, r2(seq_b2),
        r2(norm_g), r2(norm_b))

    exit_action = _exit_stage(
        sums.reshape(B, D), ex_w1, r2(ex_b1), r2(ex_lng), r2(ex_lnb), ex_w2,
        ex_b2.reshape(1, 1))

    i0, i1 = jnp.triu_indices(N_NODES, k=1)
    flat_idx = i0 * N_NODES + i1
    edge_actions = dp.reshape(B, N_NODES * N_NODES)[:, flat_idx]
    return jnp.concatenate([edge_actions, exit_action], axis=-1)


# speedup vs baseline: 20.7118x; 1.2735x over previous
import functools
import math

import jax
import jax.numpy as jnp
from jax import lax
from jax.experimental import pallas as pl
from jax.experimental.pallas import tpu as pltpu
from jax.experimental.pallas import tpu_sc as plsc

N_NODES = 128
B = 256
N = B * N_NODES
E = 524288
D = 128

NC, NS = 2, 16
SUB = 2048
NSUB_E = E // NS // SUB
NSUB = NSUB_E + 1
PER_TILE = NSUB * SUB
CHUNK_ROWS = N // (2 * NC)
CHUNK = CHUNK_ROWS * D
ROWS_PER_TILE = CHUNK_ROWS // NS
ZB = 8192


def _hist_body(edge_hbm, nid_hbm, c_hbm,
               nid_v, src_v, dst_v, bin_v, idx_v, ones_v, zero_v, shared,
               esem, ssem):
    c = lax.axis_index("c")
    s = lax.axis_index("s")
    tile_base = s * (E // NS)
    slice0 = s * (CHUNK // NS)

    def edge_fetch(b):
        ib = tile_base + b * SUB
        return (pltpu.async_copy(edge_hbm.at[0, pl.ds(ib, SUB)],
                                 src_v.at[b % 2], esem),
                pltpu.async_copy(edge_hbm.at[1, pl.ds(ib, SUB)],
                                 dst_v.at[b % 2], esem))

    edesc = edge_fetch(0)
    pltpu.sync_copy(nid_hbm, nid_v)

    z16 = jnp.zeros((16,), jnp.float32)
    o16 = jnp.ones((16,), jnp.float32)

    def zb_body(i, _):
        zero_v[pl.ds(i * 16, 16)] = z16
        return 0

    lax.fori_loop(0, ZB // 16, zb_body, 0)
    for i in range(8):
        ones_v[pl.ds(i * 16, 16)] = o16

    def zero_slice():
        descs = [pltpu.async_copy(
            zero_v, shared.at[pl.ds(slice0 + r * ZB, ZB)], esem)
            for r in range(CHUNK // NS // ZB)]
        for d in descs:
            d.wait()

    zero_slice()
    plsc.subcore_barrier()

    sdesc = [[], []]

    def fire_scatters(p):
        for r in range(SUB // 128):
            sdesc[p].append(pltpu.async_copy(
                ones_v,
                shared.at[plsc.Indices(idx_v.at[p, r], ignored_value=-1)],
                ssem, add=True))

    def drain_scatters(p):
        for d in sdesc[p]:
            d.wait()
        sdesc[p] = []

    base0 = c * 2 * CHUNK

    def unpack_nid(i16):
        w16 = plsc.load_gather(nid_v, [lax.shift_right_logical(i16, 2)])
        sh = lax.shift_left(lax.bitwise_and(i16, 3), 3)
        return lax.bitwise_and(lax.shift_right_logical(w16, sh), 127)

    def emit(p, g, bb, bin_, base):
        bin_v[pl.ds(bb + g * 16, 16)] = bin_
        rel = bin_ - base
        inr = rel.astype(jnp.uint32) < CHUNK
        idx_v[p, g // 8, pl.ds((g % 8) * 16, 16)] = jnp.where(inr, rel, -1)

    for b in range(NSUB_E):
        for d in edesc:
            d.wait()
        if b + 1 < NSUB_E:
            edesc = edge_fetch(b + 1)
        p = b % 2
        drain_scatters(p)
        bb = b * SUB

        def grp_body(g, _):
            s16 = src_v[p, pl.ds(g * 16, 16)]
            d16 = dst_v[p, pl.ds(g * 16, 16)]
            n16 = unpack_nid(s16)
            bin_ = lax.bitwise_or(lax.shift_left(d16, 7), n16)
            emit(p, g, bb, bin_, base0)
            return 0

        lax.fori_loop(0, SUB // 16, grp_body, 0)
        fire_scatters(p)

    p = NSUB_E % 2
    drain_scatters(p)
    self_base = s * (N // NS)
    lane = lax.iota(jnp.int32, 16)

    def self_body(g, _):
        i16 = lane + (self_base + g * 16)
        bin_ = lax.bitwise_or(lax.shift_left(i16, 7), unpack_nid(i16))
        emit(p, g, NSUB_E * SUB, bin_, base0)
        return 0

    lax.fori_loop(0, SUB // 16, self_body, 0)
    fire_scatters(p)

    drain_scatters(0)
    drain_scatters(1)
    plsc.subcore_barrier()
    pltpu.sync_copy(shared.at[pl.ds(slice0, CHUNK // NS)],
                    c_hbm.at[pl.ds(base0 + slice0, CHUNK // NS)])
    zero_slice()
    plsc.subcore_barrier()

    base1 = (c * 2 + 1) * CHUNK

    for b in range(NSUB):
        p = b % 2
        drain_scatters(p)
        bb = b * SUB

        def grp_body1(g, _):
            row = g // 8
            col = (g % 8) * 16
            bin_ = bin_v[pl.ds(bb + g * 16, 16)]
            rel = bin_ - base1
            inr = rel.astype(jnp.uint32) < CHUNK
            idx_v[p, row, pl.ds(col, 16)] = jnp.where(inr, rel, -1)
            return 0

        lax.fori_loop(0, SUB // 16, grp_body1, 0)
        fire_scatters(p)

    drain_scatters(0)
    drain_scatters(1)
    plsc.subcore_barrier()
    pltpu.sync_copy(shared.at[pl.ds(slice0, CHUNK // NS)],
                    c_hbm.at[pl.ds(base1 + slice0, CHUNK // NS)])


def _build_counts(edge_index, node_ids_packed):
    mesh = plsc.VectorSubcoreMesh(core_axis_name="c", subcore_axis_name="s")
    return pl.kernel(
        _hist_body,
        out_type=jax.ShapeDtypeStruct((N * D,), jnp.float32),
        mesh=mesh,
        compiler_params=pltpu.CompilerParams(needs_layout_passes=False),
        scratch_types=[
            pltpu.VMEM((N // 4,), jnp.int32),
            pltpu.VMEM((2, SUB), jnp.int32),
            pltpu.VMEM((2, SUB), jnp.int32),
            pltpu.VMEM((PER_TILE,), jnp.int32),
            pltpu.VMEM((2, SUB // 128, 128), jnp.int32),
            pltpu.VMEM((128,), jnp.float32),
            pltpu.VMEM((ZB,), jnp.float32),
            pltpu.VMEM_SHARED((CHUNK,), jnp.float32),
            pltpu.SemaphoreType.DMA,
            pltpu.SemaphoreType.DMA,
        ],
    )(edge_index, node_ids_packed)


BLK = 1024
G_PER_BLK = BLK // N_NODES
GRID = N // BLK


def _ln(h, g, b):
    m = jnp.mean(h, axis=-1, keepdims=True)
    v = jnp.mean((h - m) ** 2, axis=-1, keepdims=True)
    return (h - m) * lax.rsqrt(v + 1e-5) * g + b


def _dense_body(c_ref, emb_ref, w1_ref, b1_ref, lng_ref, lnb_ref,
                w2_ref, b2_ref, sw1_ref, sb1_ref, sw2_ref, sb2_ref,
                ng_ref, nb_ref, ew1_ref, eb1_ref, elng_ref, elnb_ref,
                ew2_ref, eb2_ref, dp_ref, exit_ref, m_s, sums_s):
    i = pl.program_id(0)

    @pl.when(i == 0)
    def _():
        m_s[:] = jnp.dot(emb_ref[:], w1_ref[:],
                         preferred_element_type=jnp.float32)

    h = jnp.dot(c_ref[:], m_s[:], preferred_element_type=jnp.float32) + b1_ref[:]
    h = _ln(h, lng_ref[:], lnb_ref[:])
    h = jnp.maximum(h, 0.0)
    h = jnp.dot(h, w2_ref[:], preferred_element_type=jnp.float32) + b2_ref[:]
    h = jnp.dot(h, sw1_ref[:], preferred_element_type=jnp.float32) + sb1_ref[:]
    h = jnp.maximum(h, 0.0)
    h = jnp.dot(h, sw2_ref[:], preferred_element_type=jnp.float32) + sb2_ref[:]
    x = _ln(h, ng_ref[:], nb_ref[:])

    scale = 1.0 / math.sqrt(D)
    for g in range(G_PER_BLK):
        xg = x[g * N_NODES:(g + 1) * N_NODES, :]
        sums_s[i * G_PER_BLK + g, :] = jnp.sum(xg, axis=0)
        dp_ref[g, :, :] = lax.dot_general(
            xg, xg, (((1,), (1,)), ((), ())),
            preferred_element_type=jnp.float32) * scale

    @pl.when(i == GRID - 1)
    def _():
        means = sums_s[:] * (1.0 / N_NODES)
        e = jnp.dot(means, ew1_ref[:],
                    preferred_element_type=jnp.float32) + eb1_ref[:]
        e = _ln(e, elng_ref[:], elnb_ref[:])
        e = jnp.maximum(e, 0.0)
        exit_ref[:] = jnp.dot(e, ew2_ref[:],
                              preferred_element_type=jnp.float32) + eb2_ref[:]


def _dense_stage(counts, emb, w1, b1, lng, lnb, w2, b2,
                 sw1, sb1, sw2, sb2, ng, nb,
                 ew1, eb1, elng, elnb, ew2, eb2):
    wspec = pl.BlockSpec((D, D), lambda i: (0, 0))
    bspec = pl.BlockSpec((1, D), lambda i: (0, 0))
    return pl.pallas_call(
        _dense_body,
        grid=(GRID,),
        in_specs=[
            pl.BlockSpec((BLK, D), lambda i: (i, 0)),
            wspec, wspec, bspec, bspec, bspec,
            wspec, bspec, wspec, bspec, wspec, bspec,
            bspec, bspec,
            wspec, bspec, bspec, bspec,
            pl.BlockSpec((D, 1), lambda i: (0, 0)),
            pl.BlockSpec((1, 1), lambda i: (0, 0)),
        ],
        out_specs=[
            pl.BlockSpec((G_PER_BLK, N_NODES, N_NODES), lambda i: (i, 0, 0)),
            pl.BlockSpec((B, 1), lambda i: (0, 0)),
        ],
        out_shape=[
            jax.ShapeDtypeStruct((B, N_NODES, N_NODES), jnp.float32),
            jax.ShapeDtypeStruct((B, 1), jnp.float32),
        ],
        scratch_shapes=[
            pltpu.VMEM((D, D), jnp.float32),
            pltpu.VMEM((B, D), jnp.float32),
        ],
    )(counts, emb, w1, b1, lng, lnb, w2, b2, sw1, sb1, sw2, sb2, ng, nb,
      ew1, eb1, elng, elnb, ew2, eb2)


def kernel(node_ids, edge_index, ptr, emb, gin_w1, gin_b1, gin_lng, gin_lnb,
           gin_w2, gin_b2, seq_w1, seq_b1, seq_w2, seq_b2, norm_g, norm_b,
           ex_w1, ex_b1, ex_lng, ex_lnb, ex_w2, ex_b2):
    del ptr
    node_ids = node_ids.astype(jnp.int32)
    nid4 = node_ids.reshape(N // 4, 4)
    nid_packed = (nid4[:, 0] | (nid4[:, 1] << 8) | (nid4[:, 2] << 16)
                  | (nid4[:, 3] << 24))

    counts = _build_counts(edge_index.astype(jnp.int32),
                           nid_packed).reshape(N, D)

    r2 = lambda v: v.reshape(1, D)
    dp, exit_action = _dense_stage(
        counts, emb, gin_w1, r2(gin_b1), r2(gin_lng), r2(gin_lnb),
        gin_w2, r2(gin_b2), seq_w1, r2(seq_b1), seq_w2, r2(seq_b2),
        r2(norm_g), r2(norm_b),
        ex_w1, r2(ex_b1), r2(ex_lng), r2(ex_lnb), ex_w2,
        ex_b2.reshape(1, 1))

    i0, i1 = jnp.triu_indices(N_NODES, k=1)
    flat_idx = i0 * N_NODES + i1
    edge_actions = dp.reshape(B, N_NODES * N_NODES)[:, flat_idx]
    return jnp.concatenate([edge_actions, exit_action], axis=-1)


# speedup vs baseline: 21.5057x; 1.0383x over previous
import functools
import math

import jax
import jax.numpy as jnp
from jax import lax
from jax.experimental import pallas as pl
from jax.experimental.pallas import tpu as pltpu
from jax.experimental.pallas import tpu_sc as plsc

N_NODES = 128
B = 256
N = B * N_NODES
E = 524288
D = 128

NC, NS = 2, 16
SUB = 2048
NSUB_E = E // NS // SUB
NSUB = NSUB_E + 1
CHUNK_ROWS = N // NC
CHUNK = CHUNK_ROWS * (D // 2)
ZB = 8192


def _hist_body(edge_hbm, nid_hbm, c_hbm,
               nid_v, src_v, dst_v, idx_v, val_v, zero_v, shared,
               esem, ssem):
    c = lax.axis_index("c")
    s = lax.axis_index("s")
    tile_base = s * (E // NS)
    slice0 = s * (CHUNK // NS)

    def edge_fetch(b):
        ib = tile_base + b * SUB
        return (pltpu.async_copy(edge_hbm.at[0, pl.ds(ib, SUB)],
                                 src_v.at[b % 2], esem),
                pltpu.async_copy(edge_hbm.at[1, pl.ds(ib, SUB)],
                                 dst_v.at[b % 2], esem))

    edesc = edge_fetch(0)
    pltpu.sync_copy(nid_hbm, nid_v)

    z16 = jnp.zeros((16,), jnp.int32)

    def zb_body(i, _):
        zero_v[pl.ds(i * 16, 16)] = z16
        return 0

    lax.fori_loop(0, ZB // 16, zb_body, 0)

    zdescs = [pltpu.async_copy(
        zero_v, shared.at[pl.ds(slice0 + r * ZB, ZB)], esem)
        for r in range(CHUNK // NS // ZB)]
    for d in zdescs:
        d.wait()
    plsc.subcore_barrier()

    sdesc = [[], []]

    def fire_scatters(p):
        for r in range(SUB // 128):
            sdesc[p].append(pltpu.async_copy(
                val_v.at[p, r],
                shared.at[plsc.Indices(idx_v.at[p, r], ignored_value=-1)],
                ssem, add=True))

    def drain_scatters(p):
        for d in sdesc[p]:
            d.wait()
        sdesc[p] = []

    base = c * CHUNK

    def unpack_nid(i16):
        w16 = plsc.load_gather(nid_v, [lax.shift_right_logical(i16, 2)])
        sh = lax.shift_left(lax.bitwise_and(i16, 3), 3)
        return lax.bitwise_and(lax.shift_right_logical(w16, sh), 127)

    def emit(p, g, d16, n16):
        widx = lax.bitwise_or(lax.shift_left(d16, 6),
                              lax.bitwise_and(n16, 63))
        rel = widx - base
        inr = rel.astype(jnp.uint32) < CHUNK
        hi4 = lax.shift_left(lax.shift_right_logical(n16, 6), 4)
        row = g // 8
        col = pl.ds((g % 8) * 16, 16)
        idx_v[p, row, col] = jnp.where(inr, rel, -1)
        val_v[p, row, col] = lax.shift_left(1, hi4)

    for b in range(NSUB_E):
        for d in edesc:
            d.wait()
        if b + 1 < NSUB_E:
            edesc = edge_fetch(b + 1)
        p = b % 2
        drain_scatters(p)

        def grp_body(g, _):
            s16 = src_v[p, pl.ds(g * 16, 16)]
            d16 = dst_v[p, pl.ds(g * 16, 16)]
            emit(p, g, d16, unpack_nid(s16))
            return 0

        lax.fori_loop(0, SUB // 16, grp_body, 0)
        fire_scatters(p)

    p = NSUB_E % 2
    drain_scatters(p)
    self_base = s * (N // NS)
    lane = lax.iota(jnp.int32, 16)

    def self_body(g, _):
        i16 = lane + (self_base + g * 16)
        emit(p, g, i16, unpack_nid(i16))
        return 0

    lax.fori_loop(0, SUB // 16, self_body, 0)
    fire_scatters(p)

    drain_scatters(0)
    drain_scatters(1)
    plsc.subcore_barrier()
    pltpu.sync_copy(shared.at[pl.ds(slice0, CHUNK // NS)],
                    c_hbm.at[pl.ds(base + slice0, CHUNK // NS)])


def _build_counts(edge_index, node_ids_packed):
    mesh = plsc.VectorSubcoreMesh(core_axis_name="c", subcore_axis_name="s")
    return pl.kernel(
        _hist_body,
        out_type=jax.ShapeDtypeStruct((N * (D // 2),), jnp.int32),
        mesh=mesh,
        compiler_params=pltpu.CompilerParams(needs_layout_passes=False),
        scratch_types=[
            pltpu.VMEM((N // 4,), jnp.int32),
            pltpu.VMEM((2, SUB), jnp.int32),
            pltpu.VMEM((2, SUB), jnp.int32),
            pltpu.VMEM((2, SUB // 128, 128), jnp.int32),
            pltpu.VMEM((2, SUB // 128, 128), jnp.int32),
            pltpu.VMEM((ZB,), jnp.int32),
            pltpu.VMEM_SHARED((CHUNK,), jnp.int32),
            pltpu.SemaphoreType.DMA,
            pltpu.SemaphoreType.DMA,
        ],
    )(edge_index, node_ids_packed)


BLK = 1024
G_PER_BLK = BLK // N_NODES
GRID = N // BLK


def _ln(h, g, b):
    m = jnp.mean(h, axis=-1, keepdims=True)
    v = jnp.mean((h - m) ** 2, axis=-1, keepdims=True)
    return (h - m) * lax.rsqrt(v + 1e-5) * g + b


def _dense_body(c_ref, emb_ref, w1_ref, b1_ref, lng_ref, lnb_ref,
                w2_ref, b2_ref, sw1_ref, sb1_ref, sw2_ref, sb2_ref,
                ng_ref, nb_ref, ew1_ref, eb1_ref, elng_ref, elnb_ref,
                ew2_ref, eb2_ref, dp_ref, exit_ref, m_s, sums_s):
    i = pl.program_id(0)

    @pl.when(i == 0)
    def _():
        m_s[:] = jnp.dot(emb_ref[:], w1_ref[:],
                         preferred_element_type=jnp.float32)

    w = c_ref[:]
    lo = lax.bitwise_and(w, 0xFFFF).astype(jnp.float32)
    hi = lax.shift_right_logical(w, 16).astype(jnp.float32)
    cnt = jnp.concatenate([lo, hi], axis=1)
    h = jnp.dot(cnt, m_s[:], preferred_element_type=jnp.float32) + b1_ref[:]
    h = _ln(h, lng_ref[:], lnb_ref[:])
    h = jnp.maximum(h, 0.0)
    h = jnp.dot(h, w2_ref[:], preferred_element_type=jnp.float32) + b2_ref[:]
    h = jnp.dot(h, sw1_ref[:], preferred_element_type=jnp.float32) + sb1_ref[:]
    h = jnp.maximum(h, 0.0)
    h = jnp.dot(h, sw2_ref[:], preferred_element_type=jnp.float32) + sb2_ref[:]
    x = _ln(h, ng_ref[:], nb_ref[:])

    scale = 1.0 / math.sqrt(D)
    for g in range(G_PER_BLK):
        xg = x[g * N_NODES:(g + 1) * N_NODES, :]
        sums_s[i * G_PER_BLK + g, :] = jnp.sum(xg, axis=0)
        dp_ref[g, :, :] = lax.dot_general(
            xg, xg, (((1,), (1,)), ((), ())),
            preferred_element_type=jnp.float32) * scale

    @pl.when(i == GRID - 1)
    def _():
        means = sums_s[:] * (1.0 / N_NODES)
        e = jnp.dot(means, ew1_ref[:],
                    preferred_element_type=jnp.float32) + eb1_ref[:]
        e = _ln(e, elng_ref[:], elnb_ref[:])
        e = jnp.maximum(e, 0.0)
        exit_ref[:] = jnp.dot(e, ew2_ref[:],
                              preferred_element_type=jnp.float32) + eb2_ref[:]


def _dense_stage(counts, emb, w1, b1, lng, lnb, w2, b2,
                 sw1, sb1, sw2, sb2, ng, nb,
                 ew1, eb1, elng, elnb, ew2, eb2):
    wspec = pl.BlockSpec((D, D), lambda i: (0, 0))
    bspec = pl.BlockSpec((1, D), lambda i: (0, 0))
    return pl.pallas_call(
        _dense_body,
        grid=(GRID,),
        in_specs=[
            pl.BlockSpec((BLK, D // 2), lambda i: (i, 0)),
            wspec, wspec, bspec, bspec, bspec,
            wspec, bspec, wspec, bspec, wspec, bspec,
            bspec, bspec,
            wspec, bspec, bspec, bspec,
            pl.BlockSpec((D, 1), lambda i: (0, 0)),
            pl.BlockSpec((1, 1), lambda i: (0, 0)),
        ],
        out_specs=[
            pl.BlockSpec((G_PER_BLK, N_NODES, N_NODES), lambda i: (i, 0, 0)),
            pl.BlockSpec((B, 1), lambda i: (0, 0)),
        ],
        out_shape=[
            jax.ShapeDtypeStruct((B, N_NODES, N_NODES), jnp.float32),
            jax.ShapeDtypeStruct((B, 1), jnp.float32),
        ],
        scratch_shapes=[
            pltpu.VMEM((D, D), jnp.float32),
            pltpu.VMEM((B, D), jnp.float32),
        ],
    )(counts, emb, w1, b1, lng, lnb, w2, b2, sw1, sb1, sw2, sb2, ng, nb,
      ew1, eb1, elng, elnb, ew2, eb2)


def kernel(node_ids, edge_index, ptr, emb, gin_w1, gin_b1, gin_lng, gin_lnb,
           gin_w2, gin_b2, seq_w1, seq_b1, seq_w2, seq_b2, norm_g, norm_b,
           ex_w1, ex_b1, ex_lng, ex_lnb, ex_w2, ex_b2):
    del ptr
    node_ids = node_ids.astype(jnp.int32)
    nid4 = node_ids.reshape(N // 4, 4)
    nid_packed = (nid4[:, 0] | (nid4[:, 1] << 8) | (nid4[:, 2] << 16)
                  | (nid4[:, 3] << 24))

    counts = _build_counts(edge_index.astype(jnp.int32),
                           nid_packed).reshape(N, D // 2)

    r2 = lambda v: v.reshape(1, D)
    dp, exit_action = _dense_stage(
        counts, emb, gin_w1, r2(gin_b1), r2(gin_lng), r2(gin_lnb),
        gin_w2, r2(gin_b2), seq_w1, r2(seq_b1), seq_w2, r2(seq_b2),
        r2(norm_g), r2(norm_b),
        ex_w1, r2(ex_b1), r2(ex_lng), r2(ex_lnb), ex_w2,
        ex_b2.reshape(1, 1))

    i0, i1 = jnp.triu_indices(N_NODES, k=1)
    flat_idx = i0 * N_NODES + i1
    edge_actions = dp.reshape(B, N_NODES * N_NODES)[:, flat_idx]
    return jnp.concatenate([edge_actions, exit_action], axis=-1)
